# Initial kernel scaffold; baseline (speedup 1.0000x reference)
import jax
import jax.numpy as jnp
from jax.experimental import pallas as pl


def kernel(x_c, adj, mode, params):
    raise NotImplementedError("write your pallas kernel here")


# speedup vs baseline: 1.5488x; 1.5488x over previous
import math

import numpy as np
import jax
import jax.numpy as jnp
from jax.experimental import pallas as pl

K = 10
D = 64
DFF = 256
H = 4
DH = 16
T = 12
NB = 256
NL = 2


def _make_pe(t, d):
    pos = np.arange(t)[:, None].astype(np.float64)
    div = np.exp(np.arange(0, d, 2).astype(np.float64) * (-np.log(10000.0) / d))
    pe = np.zeros((t, d))
    pe[:, 0::2] = np.sin(pos * div)
    pe[:, 1::2] = np.cos(pos * div)
    return pe.astype(np.float32)


def _make_attn_mats():
    seg = np.zeros((T * D, H * T), np.float32)
    expm = np.zeros((H * T, T * D), np.float32)
    red = np.zeros((T * D, D), np.float32)
    for j in range(T):
        for h in range(H):
            for d in range(DH):
                lane = j * D + h * DH + d
                seg[lane, h * T + j] = 1.0
                expm[h * T + j, lane] = 1.0
                red[lane, h * DH + d] = 1.0
    return seg, expm, red


_PE = _make_pe(T, D)
_SEG, _EXPM, _RED = _make_attn_mats()

_PNAMES = []
for _pre, _natt in (("enc", 1), ("dec", 2)):
    for _a in range(_natt):
        for _nm in ("Wq", "Wk", "Wv", "Wo", "bq", "bk", "bv", "bo", "ln_g", "ln_b"):
            _PNAMES.append(_pre + str(_a) + "_" + _nm)
    for _nm in ("W1", "b1", "W2", "b2", "ln_g", "ln_b"):
        _PNAMES.append(_pre + "_ffn_" + _nm)
    _PNAMES.append(_pre + "_fin_g")
    _PNAMES.append(_pre + "_fin_b")
_PNAMES.append("gen_W")
_PNAMES.append("gen_b")


def _prep_kernel(adj_ref, sx_full_ref, sx_blk_ref, pe_ref, srcw_ref, srcb_ref,
                 tgtw_ref, tgtb_ref, se_ref, te_ref):
    a = adj_ref[0]
    sx = sx_full_ref[0]
    own = sx_blk_ref[0]
    n = a.shape[1]
    iota = jax.lax.broadcasted_iota(jnp.int32, (NB, n), 1)
    sels = []
    acc = own
    for _ in range(K):
        m = jnp.max(a, axis=1, keepdims=True)
        ismax = a == m
        idx = jnp.min(jnp.where(ismax, iota, n), axis=1, keepdims=True)
        fm = iota == idx
        sel = jnp.dot(fm.astype(jnp.float32), sx,
                      preferred_element_type=jnp.float32)
        sels.append(sel)
        acc = acc + sel
        a = jnp.where(fm, -jnp.inf, a)
    tgt = acc * (1.0 / (K + 1))
    chans = [own] + sels
    scale = math.sqrt(D)
    srcw = srcw_ref[...]
    srcb = srcb_ref[...]
    tgtw = tgtw_ref[...]
    tgtb = tgtb_ref[...]
    pe = pe_ref[...]
    se_rows = []
    te_rows = []
    for t in range(T):
        s = srcb
        for c in range(K + 1):
            s = s + chans[c][:, t:t + 1] * srcw[c:c + 1, :]
        se_rows.append(s * scale + pe[t:t + 1, :])
        tt = tgt[:, t:t + 1] * tgtw + tgtb
        te_rows.append(tt * scale + pe[t:t + 1, :])
    se_ref[0] = jnp.concatenate(se_rows, axis=0)
    te_ref[0] = jnp.concatenate(te_rows, axis=0)


def _ln(x, g, b):
    m = jnp.mean(x, axis=1, keepdims=True)
    v = jnp.mean((x - m) ** 2, axis=1, keepdims=True)
    return (x - m) / jnp.sqrt(v + 1e-6) * g + b


def _lane_stack(x):
    return jnp.concatenate([x[t * NB:(t + 1) * NB, :] for t in range(T)],
                           axis=1)


def _mm(a, b):
    return jnp.dot(a, b, preferred_element_type=jnp.float32)


def _attn(xq, xkv, wq, bq, wk, bk, wv, bv, wo, bo, seg, expm, red, causal):
    q = _mm(xq, wq) + bq
    k = _mm(xkv, wk) + bk
    v = _mm(xkv, wv) + bv
    kc = _lane_stack(k)
    vc = _lane_stack(v)
    inv = 1.0 / math.sqrt(DH)
    outs = []
    for i in range(T):
        qi = q[i * NB:(i + 1) * NB, :]
        qt = jnp.concatenate([qi] * T, axis=1)
        sc = _mm(qt * kc, seg) * inv
        parts = []
        for h in range(H):
            s = sc[:, h * T:(h + 1) * T]
            if causal:
                jidx = jax.lax.broadcasted_iota(jnp.int32, (NB, T), 1)
                s = jnp.where(jidx <= i, s, -1e9)
            mx = jnp.max(s, axis=1, keepdims=True)
            e = jnp.exp(s - mx)
            parts.append(e / jnp.sum(e, axis=1, keepdims=True))
        att = jnp.concatenate(parts, axis=1)
        ae = _mm(att, expm)
        outs.append(_mm(ae * vc, red))
    o = jnp.concatenate(outs, axis=0)
    return _mm(o, wo) + bo


def _ffn(x, w1, b1, w2, b2):
    hdn = jnp.maximum(_mm(x, w1) + b1, 0.0)
    return _mm(hdn, w2) + b2


def _tform_kernel(se_ref, te_ref, seg_ref, expm_ref, red_ref, *rest):
    out_ref = rest[-1]
    p = {name: r for name, r in zip(_PNAMES, rest[:-1])}
    seg = seg_ref[...]
    expm = expm_ref[...]
    red = red_ref[...]

    def w3(name, l):
        return p[name][l]

    def w2(name, l):
        return p[name][l:l + 1, :]

    x = se_ref[0]
    for l in range(NL):
        y = _ln(x, w2("enc0_ln_g", l), w2("enc0_ln_b", l))
        x = x + _attn(y, y, w3("enc0_Wq", l), w2("enc0_bq", l),
                      w3("enc0_Wk", l), w2("enc0_bk", l),
                      w3("enc0_Wv", l), w2("enc0_bv", l),
                      w3("enc0_Wo", l), w2("enc0_bo", l),
                      seg, expm, red, False)
        y = _ln(x, w2("enc_ffn_ln_g", l), w2("enc_ffn_ln_b", l))
        x = x + _ffn(y, w3("enc_ffn_W1", l), w2("enc_ffn_b1", l),
                     w3("enc_ffn_W2", l), w2("enc_ffn_b2", l))
    mem = _ln(x, p["enc_fin_g"][...], p["enc_fin_b"][...])

    x = te_ref[0]
    for l in range(NL):
        y = _ln(x, w2("dec0_ln_g", l), w2("dec0_ln_b", l))
        x = x + _attn(y, y, w3("dec0_Wq", l), w2("dec0_bq", l),
                      w3("dec0_Wk", l), w2("dec0_bk", l),
                      w3("dec0_Wv", l), w2("dec0_bv", l),
                      w3("dec0_Wo", l), w2("dec0_bo", l),
                      seg, expm, red, True)
        y = _ln(x, w2("dec1_ln_g", l), w2("dec1_ln_b", l))
        x = x + _attn(y, mem, w3("dec1_Wq", l), w2("dec1_bq", l),
                      w3("dec1_Wk", l), w2("dec1_bk", l),
                      w3("dec1_Wv", l), w2("dec1_bv", l),
                      w3("dec1_Wo", l), w2("dec1_bo", l),
                      seg, expm, red, False)
        y = _ln(x, w2("dec_ffn_ln_g", l), w2("dec_ffn_ln_b", l))
        x = x + _ffn(y, w3("dec_ffn_W1", l), w2("dec_ffn_b1", l),
                     w3("dec_ffn_W2", l), w2("dec_ffn_b2", l))
    y = _ln(x, p["dec_fin_g"][...], p["dec_fin_b"][...])

    r = jnp.sum(y * p["gen_W"][...], axis=1, keepdims=True) \
        + p["gen_b"][...]
    out_ref[0] = jnp.concatenate(
        [r[t * NB:(t + 1) * NB, :] for t in range(T)], axis=1)


def kernel(x_c, adj, mode, params):
    bs, t_len, _, n = x_c.shape
    nblk = n // NB
    nblocks = bs * nblk
    sx0 = jnp.transpose(x_c[:, :, 0, :], (0, 2, 1))

    pe = jnp.asarray(_PE)
    srcw = params["src_W"]
    srcb = params["src_b"].reshape(1, D)
    tgtw = params["tgt_W"]
    tgtb = params["tgt_b"].reshape(1, D)

    f32 = jnp.float32
    se, te = pl.pallas_call(
        _prep_kernel,
        grid=(bs, nblk),
        in_specs=[
            pl.BlockSpec((1, NB, n), lambda b, i: (b, i, 0)),
            pl.BlockSpec((1, n, T), lambda b, i: (b, 0, 0)),
            pl.BlockSpec((1, NB, T), lambda b, i: (b, i, 0)),
            pl.BlockSpec((T, D), lambda b, i: (0, 0)),
            pl.BlockSpec((K + 1, D), lambda b, i: (0, 0)),
            pl.BlockSpec((1, D), lambda b, i: (0, 0)),
            pl.BlockSpec((1, D), lambda b, i: (0, 0)),
            pl.BlockSpec((1, D), lambda b, i: (0, 0)),
        ],
        out_specs=[
            pl.BlockSpec((1, T * NB, D), lambda b, i: (b * nblk + i, 0, 0)),
            pl.BlockSpec((1, T * NB, D), lambda b, i: (b * nblk + i, 0, 0)),
        ],
        out_shape=[
            jax.ShapeDtypeStruct((nblocks, T * NB, D), f32),
            jax.ShapeDtypeStruct((nblocks, T * NB, D), f32),
        ],
    )(adj, sx0, sx0, pe, srcw, srcb, tgtw, tgtb)

    plist = []
    for name in _PNAMES:
        arr = params[name]
        if arr.ndim == 1:
            arr = arr.reshape(1, -1)
        elif name == "gen_W":
            arr = arr.reshape(1, D)
        plist.append(arr)

    full = lambda a: pl.BlockSpec(a.shape, lambda i: (0,) * a.ndim)
    out = pl.pallas_call(
        _tform_kernel,
        grid=(nblocks,),
        in_specs=[
            pl.BlockSpec((1, T * NB, D), lambda i: (i, 0, 0)),
            pl.BlockSpec((1, T * NB, D), lambda i: (i, 0, 0)),
            full(_SEG), full(_EXPM), full(_RED),
        ] + [full(a) for a in plist],
        out_specs=pl.BlockSpec((1, NB, T), lambda i: (i, 0, 0)),
        out_shape=jax.ShapeDtypeStruct((nblocks, NB, T), f32),
    )(se, te, jnp.asarray(_SEG), jnp.asarray(_EXPM), jnp.asarray(_RED),
      *plist)

    return out.reshape(bs, n, t_len)
